# baseline (speedup 1.0000x reference)
import functools

import jax
import jax.numpy as jnp
from jax import lax
from jax.experimental import pallas as pl
from jax.experimental.pallas import tpu as pltpu
from jax.experimental.pallas import tpu_sc as plsc

N_ATOMS = 50000
HIDDEN = 128
MAX_NB = 12
ATOM_FDIM = 133
FEAT_DIM = 200
D_FF = 512
FFN_HID = 512
N_TASKS = 12
MOL_SIZE = 25
N_MOLS = 2000

NW = 32
BPW = 1568
PAD_ATOMS = NW * BPW
CHUNK = 8
NIDX = CHUNK * MAX_NB
CPW = BPW // CHUNK
NBUF = 7

MOLS_PER_BLK = 80
BLK_ROWS = MOLS_PER_BLK * MOL_SIZE
N_BLKS = N_MOLS // MOLS_PER_BLK


def _gather_sum_sc(table, idx_rows):
    mesh = plsc.VectorSubcoreMesh(core_axis_name="c", subcore_axis_name="s")

    @functools.partial(
        pl.kernel,
        mesh=mesh,
        out_type=jax.ShapeDtypeStruct((PAD_ATOMS, HIDDEN), jnp.float32),
        scratch_types=[
            pltpu.VMEM((CPW, NIDX), jnp.int32),
            pltpu.VMEM((NBUF, NIDX, HIDDEN), jnp.float32),
            pltpu.VMEM((NBUF, CHUNK, HIDDEN), jnp.float32),
        ] + [pltpu.SemaphoreType.DMA] * (2 * NBUF),
    )
    def k(table_hbm, idx_hbm, out_hbm, idx_v, rbufs, obufs, *sems):
        isems = sems[:NBUF]
        osems = sems[NBUF:]
        wid = lax.axis_index("s") * 2 + lax.axis_index("c")
        abase = wid * BPW
        pltpu.sync_copy(idx_hbm.at[wid], idx_v)
        for b in range(NBUF):
            pltpu.make_async_copy(
                table_hbm.at[idx_v.at[b]], rbufs.at[b], isems[b]).start()

        def ring(i, carry):
            for b in range(NBUF):
                g = NBUF * i + b
                rb = rbufs.at[b]
                ov = obufs.at[b]
                pltpu.make_async_copy(
                    table_hbm.at[idx_v.at[g]], rb, isems[b]).wait()

                @pl.when(g >= NBUF)
                def _():
                    pltpu.make_async_copy(
                        ov,
                        out_hbm.at[pl.ds(abase + (g - NBUF) * CHUNK, CHUNK)],
                        osems[b]).wait()

                def atom(a, c2):
                    r = a * MAX_NB
                    for c in range(HIDDEN // 16):
                        col = pl.ds(c * 16, 16)
                        acc = rb[r, col]
                        for j in range(1, MAX_NB):
                            acc = acc + rb[r + j, col]
                        ov[a, col] = acc
                    return c2

                lax.fori_loop(0, CHUNK, atom, 0)
                pltpu.make_async_copy(
                    ov, out_hbm.at[pl.ds(abase + g * CHUNK, CHUNK)],
                    osems[b]).start()

                @pl.when(g + NBUF < CPW)
                def _():
                    pltpu.make_async_copy(
                        table_hbm.at[idx_v.at[g + NBUF]], rb, isems[b]).start()
            return carry

        lax.fori_loop(0, CPW // NBUF, ring, 0)
        for b in range(NBUF):
            g = CPW - NBUF + b
            pltpu.make_async_copy(
                obufs.at[b], out_hbm.at[pl.ds(abase + g * CHUNK, CHUNK)],
                osems[b]).wait()

    return k(table, idx_rows)


def _ffn_pool_body(x1_ref, x2_ref, w1f_ref, w1s_ref, b1_ref, w2_ref, b2_ref,
                   g_ref, bb_ref, out_ref):
    x1 = x1_ref[...].astype(jnp.bfloat16)
    x2 = x2_ref[...].astype(jnp.bfloat16)
    h = jnp.dot(x1, w1f_ref[...], preferred_element_type=jnp.float32)
    h = h + jnp.dot(x2, w1s_ref[...], preferred_element_type=jnp.float32)
    h = jnp.maximum(h + b1_ref[...], 0.0).astype(jnp.bfloat16)
    y = jnp.dot(h, w2_ref[...], preferred_element_type=jnp.float32) + b2_ref[...]
    mu = jnp.mean(y, axis=-1, keepdims=True)
    var = jnp.mean(jnp.square(y - mu), axis=-1, keepdims=True)
    yn = (y - mu) * lax.rsqrt(var + 1e-5) * g_ref[...] + bb_ref[...]
    rid = lax.broadcasted_iota(jnp.int32, (MOLS_PER_BLK, BLK_ROWS), 0)
    cid = lax.broadcasted_iota(jnp.int32, (MOLS_PER_BLK, BLK_ROWS), 1)
    sel = jnp.where(cid // MOL_SIZE == rid, 1.0 / MOL_SIZE, 0.0)
    out_ref[...] = jnp.dot(sel, yn, preferred_element_type=jnp.float32)


def _ffn_pool_tc(f_atoms, aggr, w1f, w1s, b1, w2, b2, g, bb):
    return pl.pallas_call(
        _ffn_pool_body,
        grid=(N_BLKS,),
        in_specs=[
            pl.BlockSpec((BLK_ROWS, ATOM_FDIM), lambda i: (i, 0)),
            pl.BlockSpec((BLK_ROWS, HIDDEN), lambda i: (i, 0)),
            pl.BlockSpec((ATOM_FDIM, D_FF), lambda i: (0, 0)),
            pl.BlockSpec((HIDDEN, D_FF), lambda i: (0, 0)),
            pl.BlockSpec((1, D_FF), lambda i: (0, 0)),
            pl.BlockSpec((D_FF, HIDDEN), lambda i: (0, 0)),
            pl.BlockSpec((1, HIDDEN), lambda i: (0, 0)),
            pl.BlockSpec((1, HIDDEN), lambda i: (0, 0)),
            pl.BlockSpec((1, HIDDEN), lambda i: (0, 0)),
        ],
        out_specs=pl.BlockSpec((MOLS_PER_BLK, HIDDEN), lambda i: (i, 0)),
        out_shape=jax.ShapeDtypeStruct((N_MOLS, HIDDEN), jnp.float32),
    )(f_atoms, aggr, w1f, w1s, b1, w2, b2, g, bb)


def _mol_ffn_body(p_ref, f_ref, wp_ref, wf_ref, b1_ref, w2_ref, b2_ref, o_ref):
    h = jnp.dot(p_ref[...], wp_ref[...], preferred_element_type=jnp.float32)
    h = h + jnp.dot(f_ref[...], wf_ref[...], preferred_element_type=jnp.float32)
    h = jnp.maximum(h + b1_ref[...], 0.0)
    o = jnp.dot(h, w2_ref[...], preferred_element_type=jnp.float32) + b2_ref[...]
    o_ref[...] = jax.nn.sigmoid(o) * 0.5


def _mol_ffn_tc(pooled, features, wp, wf, b1, w2, b2):
    return pl.pallas_call(
        _mol_ffn_body,
        out_shape=jax.ShapeDtypeStruct((N_MOLS, N_TASKS), jnp.float32),
    )(pooled, features, wp, wf, b1, w2, b2)


def kernel(atom_output, bond_output, original_f_atoms, original_f_bonds, a2a,
           a2b, b2a, b2revb, a_scope, features_batch, W1a, b1a, W2a, b2a_w,
           ln1_g, ln1_b, W1b, b1b, W2b, b2b_w, ln2_g, ln2_b, Wm1, bm1, Wm2,
           bm2):
    idx_flat = a2a.reshape(-1)
    idx_flat = jnp.pad(idx_flat, (0, (PAD_ATOMS - N_ATOMS) * MAX_NB))
    idx_rows = idx_flat.reshape(NW, CPW, NIDX)

    aggr = _gather_sum_sc(atom_output, idx_rows)[:N_ATOMS]

    pooled = _ffn_pool_tc(
        original_f_atoms, aggr,
        W1a[:ATOM_FDIM].astype(jnp.bfloat16),
        W1a[ATOM_FDIM:].astype(jnp.bfloat16),
        b1a.reshape(1, D_FF),
        W2a.astype(jnp.bfloat16), b2a_w.reshape(1, HIDDEN),
        ln1_g.reshape(1, HIDDEN), ln1_b.reshape(1, HIDDEN))

    out = _mol_ffn_tc(
        pooled, features_batch,
        Wm1[:HIDDEN], Wm1[HIDDEN:], bm1.reshape(1, FFN_HID),
        Wm2, bm2.reshape(1, N_TASKS))
    return out


# speedup vs baseline: 1.0077x; 1.0077x over previous
import functools

import jax
import jax.numpy as jnp
from jax import lax
from jax.experimental import pallas as pl
from jax.experimental.pallas import tpu as pltpu
from jax.experimental.pallas import tpu_sc as plsc

N_ATOMS = 50000
HIDDEN = 128
MAX_NB = 12
ATOM_FDIM = 133
FEAT_DIM = 200
D_FF = 512
FFN_HID = 512
N_TASKS = 12
MOL_SIZE = 25
N_MOLS = 2000

NW = 32
BPW = 1568
PAD_ATOMS = NW * BPW
CHUNK = 8
NIDX = CHUNK * MAX_NB
CPW = BPW // CHUNK
NBUF = 4

MOLS_PER_BLK = 80
BLK_ROWS = MOLS_PER_BLK * MOL_SIZE
N_BLKS = N_MOLS // MOLS_PER_BLK


def _gather_sum_sc(table, idx_rows):
    mesh = plsc.VectorSubcoreMesh(core_axis_name="c", subcore_axis_name="s")

    @functools.partial(
        pl.kernel,
        mesh=mesh,
        out_type=jax.ShapeDtypeStruct((PAD_ATOMS, HIDDEN), jnp.float32),
        scratch_types=[
            pltpu.VMEM((CPW, NIDX), jnp.int32),
            pltpu.VMEM((NBUF, NIDX, HIDDEN), jnp.float32),
            pltpu.VMEM((NBUF, CHUNK, HIDDEN), jnp.float32),
        ] + [pltpu.SemaphoreType.DMA] * (2 * NBUF),
    )
    def k(table_hbm, idx_hbm, out_hbm, idx_v, rbufs, obufs, *sems):
        isems = sems[:NBUF]
        osems = sems[NBUF:]
        wid = lax.axis_index("s") * 2 + lax.axis_index("c")
        abase = wid * BPW
        pltpu.sync_copy(idx_hbm.at[wid], idx_v)
        for b in range(NBUF):
            pltpu.make_async_copy(
                table_hbm.at[idx_v.at[b]], rbufs.at[b], isems[b]).start()

        def ring(i, carry):
            for b in range(NBUF):
                g = NBUF * i + b
                rb = rbufs.at[b]
                ov = obufs.at[b]
                pltpu.make_async_copy(
                    table_hbm.at[idx_v.at[g]], rb, isems[b]).wait()

                @pl.when(g >= NBUF)
                def _():
                    pltpu.make_async_copy(
                        ov,
                        out_hbm.at[pl.ds(abase + (g - NBUF) * CHUNK, CHUNK)],
                        osems[b]).wait()

                def atom(a, c2):
                    r = a * MAX_NB
                    for c in range(HIDDEN // 16):
                        col = pl.ds(c * 16, 16)
                        acc = rb[r, col]
                        for j in range(1, MAX_NB):
                            acc = acc + rb[r + j, col]
                        ov[a, col] = acc
                    return c2

                lax.fori_loop(0, CHUNK, atom, 0)
                pltpu.make_async_copy(
                    ov, out_hbm.at[pl.ds(abase + g * CHUNK, CHUNK)],
                    osems[b]).start()

                @pl.when(g + NBUF < CPW)
                def _():
                    pltpu.make_async_copy(
                        table_hbm.at[idx_v.at[g + NBUF]], rb, isems[b]).start()
            return carry

        lax.fori_loop(0, CPW // NBUF, ring, 0)
        for b in range(NBUF):
            g = CPW - NBUF + b
            pltpu.make_async_copy(
                obufs.at[b], out_hbm.at[pl.ds(abase + g * CHUNK, CHUNK)],
                osems[b]).wait()

    return k(table, idx_rows)


def _ffn_pool_body(x1_ref, x2_ref, w1f_ref, w1s_ref, b1_ref, w2_ref, b2_ref,
                   g_ref, bb_ref, f_ref, wp_ref, wf_ref, bm1_ref, wm2_ref,
                   bm2_ref, out_ref, pooled_scr):
    i = pl.program_id(0)
    x1 = x1_ref[...].astype(jnp.bfloat16)
    x2 = x2_ref[...].astype(jnp.bfloat16)
    h = jnp.dot(x1, w1f_ref[...], preferred_element_type=jnp.float32)
    h = h + jnp.dot(x2, w1s_ref[...], preferred_element_type=jnp.float32)
    h = jnp.maximum(h + b1_ref[...], 0.0).astype(jnp.bfloat16)
    y = jnp.dot(h, w2_ref[...], preferred_element_type=jnp.float32) + b2_ref[...]
    mu = jnp.mean(y, axis=-1, keepdims=True)
    var = jnp.mean(jnp.square(y - mu), axis=-1, keepdims=True)
    yn = (y - mu) * lax.rsqrt(var + 1e-5) * g_ref[...] + bb_ref[...]
    rid = lax.broadcasted_iota(jnp.int32, (MOLS_PER_BLK, BLK_ROWS), 0)
    cid = lax.broadcasted_iota(jnp.int32, (MOLS_PER_BLK, BLK_ROWS), 1)
    sel = jnp.where(cid // MOL_SIZE == rid, 1.0 / MOL_SIZE, 0.0)
    pooled_scr[pl.ds(i * MOLS_PER_BLK, MOLS_PER_BLK), :] = jnp.dot(
        sel, yn, preferred_element_type=jnp.float32)

    @pl.when(i == N_BLKS - 1)
    def _():
        p = pooled_scr[...]
        hm = jnp.dot(p, wp_ref[...], preferred_element_type=jnp.float32)
        hm = hm + jnp.dot(f_ref[...], wf_ref[...],
                          preferred_element_type=jnp.float32)
        hm = jnp.maximum(hm + bm1_ref[...], 0.0)
        o = jnp.dot(hm, wm2_ref[...],
                    preferred_element_type=jnp.float32) + bm2_ref[...]
        out_ref[...] = jax.nn.sigmoid(o) * 0.5


def _ffn_pool_tc(f_atoms, aggr, w1f, w1s, b1, w2, b2, g, bb, features, wp, wf,
                 bm1, wm2, bm2):
    return pl.pallas_call(
        _ffn_pool_body,
        grid=(N_BLKS,),
        in_specs=[
            pl.BlockSpec((BLK_ROWS, ATOM_FDIM), lambda i: (i, 0)),
            pl.BlockSpec((BLK_ROWS, HIDDEN), lambda i: (i, 0)),
            pl.BlockSpec((ATOM_FDIM, D_FF), lambda i: (0, 0)),
            pl.BlockSpec((HIDDEN, D_FF), lambda i: (0, 0)),
            pl.BlockSpec((1, D_FF), lambda i: (0, 0)),
            pl.BlockSpec((D_FF, HIDDEN), lambda i: (0, 0)),
            pl.BlockSpec((1, HIDDEN), lambda i: (0, 0)),
            pl.BlockSpec((1, HIDDEN), lambda i: (0, 0)),
            pl.BlockSpec((1, HIDDEN), lambda i: (0, 0)),
            pl.BlockSpec((N_MOLS, FEAT_DIM), lambda i: (0, 0)),
            pl.BlockSpec((HIDDEN, FFN_HID), lambda i: (0, 0)),
            pl.BlockSpec((FEAT_DIM, FFN_HID), lambda i: (0, 0)),
            pl.BlockSpec((1, FFN_HID), lambda i: (0, 0)),
            pl.BlockSpec((FFN_HID, N_TASKS), lambda i: (0, 0)),
            pl.BlockSpec((1, N_TASKS), lambda i: (0, 0)),
        ],
        out_specs=pl.BlockSpec((N_MOLS, N_TASKS), lambda i: (0, 0)),
        out_shape=jax.ShapeDtypeStruct((N_MOLS, N_TASKS), jnp.float32),
        scratch_shapes=[pltpu.VMEM((N_MOLS, HIDDEN), jnp.float32)],
    )(f_atoms, aggr, w1f, w1s, b1, w2, b2, g, bb, features, wp, wf, bm1, wm2,
      bm2)


def kernel(atom_output, bond_output, original_f_atoms, original_f_bonds, a2a,
           a2b, b2a, b2revb, a_scope, features_batch, W1a, b1a, W2a, b2a_w,
           ln1_g, ln1_b, W1b, b1b, W2b, b2b_w, ln2_g, ln2_b, Wm1, bm1, Wm2,
           bm2):
    idx_flat = a2a.reshape(-1)
    idx_flat = jnp.pad(idx_flat, (0, (PAD_ATOMS - N_ATOMS) * MAX_NB))
    idx_rows = idx_flat.reshape(NW, CPW, NIDX)

    aggr = _gather_sum_sc(atom_output, idx_rows)[:N_ATOMS]

    out = _ffn_pool_tc(
        original_f_atoms, aggr,
        W1a[:ATOM_FDIM].astype(jnp.bfloat16),
        W1a[ATOM_FDIM:].astype(jnp.bfloat16),
        b1a.reshape(1, D_FF),
        W2a.astype(jnp.bfloat16), b2a_w.reshape(1, HIDDEN),
        ln1_g.reshape(1, HIDDEN), ln1_b.reshape(1, HIDDEN),
        features_batch,
        Wm1[:HIDDEN], Wm1[HIDDEN:], bm1.reshape(1, FFN_HID),
        Wm2, bm2.reshape(1, N_TASKS))
    return out


# speedup vs baseline: 1.0478x; 1.0398x over previous
import functools

import jax
import jax.numpy as jnp
from jax import lax
from jax.experimental import pallas as pl
from jax.experimental.pallas import tpu as pltpu
from jax.experimental.pallas import tpu_sc as plsc

N_ATOMS = 50000
HIDDEN = 128
MAX_NB = 12
ATOM_FDIM = 133
FEAT_DIM = 200
D_FF = 512
FFN_HID = 512
N_TASKS = 12
MOL_SIZE = 25
N_MOLS = 2000

NW = 32
BPW = 1568
PAD_ATOMS = NW * BPW
CHUNK = 8
NIDX = CHUNK * MAX_NB
CPW = BPW // CHUNK
NBUF = 4

MOLS_PER_BLK = 80
BLK_ROWS = MOLS_PER_BLK * MOL_SIZE
N_BLKS = N_MOLS // MOLS_PER_BLK


def _gather_sum_sc(table, idx_rows):
    mesh = plsc.VectorSubcoreMesh(core_axis_name="c", subcore_axis_name="s")

    @functools.partial(
        pl.kernel,
        mesh=mesh,
        out_type=jax.ShapeDtypeStruct((PAD_ATOMS, HIDDEN), jnp.float32),
        scratch_types=[
            pltpu.VMEM((CPW, NIDX), jnp.int32),
            pltpu.VMEM((NBUF, NIDX, HIDDEN), jnp.float32),
            pltpu.VMEM((NBUF, CHUNK, HIDDEN), jnp.float32),
        ] + [pltpu.SemaphoreType.DMA] * (2 * NBUF),
    )
    def k(table_hbm, idx_hbm, out_hbm, idx_v, rbufs, obufs, *sems):
        isems = sems[:NBUF]
        osems = sems[NBUF:]
        wid = lax.axis_index("s") * 2 + lax.axis_index("c")
        abase = wid * BPW
        pltpu.sync_copy(idx_hbm.at[wid], idx_v)
        for b in range(NBUF):
            pltpu.make_async_copy(
                table_hbm.at[idx_v.at[b]], rbufs.at[b], isems[b]).start()

        def ring(i, carry):
            for b in range(NBUF):
                g = NBUF * i + b
                rb = rbufs.at[b]
                ov = obufs.at[b]
                pltpu.make_async_copy(
                    table_hbm.at[idx_v.at[g]], rb, isems[b]).wait()

                @pl.when(g >= NBUF)
                def _():
                    pltpu.make_async_copy(
                        ov,
                        out_hbm.at[pl.ds(abase + (g - NBUF) * CHUNK, CHUNK)],
                        osems[b]).wait()

                def atom(a, c2):
                    r = a * MAX_NB
                    for c in range(HIDDEN // 16):
                        col = pl.ds(c * 16, 16)
                        acc = rb[r, col]
                        for j in range(1, MAX_NB):
                            acc = acc + rb[r + j, col]
                        ov[a, col] = acc
                    return c2

                lax.fori_loop(0, CHUNK, atom, 0)
                pltpu.make_async_copy(
                    ov, out_hbm.at[pl.ds(abase + g * CHUNK, CHUNK)],
                    osems[b]).start()

                @pl.when(g + NBUF < CPW)
                def _():
                    pltpu.make_async_copy(
                        table_hbm.at[idx_v.at[g + NBUF]], rb, isems[b]).start()
            return carry

        lax.fori_loop(0, CPW // NBUF, ring, 0)
        for b in range(NBUF):
            g = CPW - NBUF + b
            pltpu.make_async_copy(
                obufs.at[b], out_hbm.at[pl.ds(abase + g * CHUNK, CHUNK)],
                osems[b]).wait()

    return k(table, idx_rows)


def _ffn_pool_body(x1_ref, x2_ref, w1f_ref, w1s_ref, b1_ref, w2_ref, b2_ref,
                   g_ref, bb_ref, f_ref, wp_ref, wf_ref, bm1_ref, wm2_ref,
                   bm2_ref, out_ref, pooled_scr):
    i = pl.program_id(0)
    x1 = x1_ref[...].astype(jnp.bfloat16)
    x2 = x2_ref[...].astype(jnp.bfloat16)
    h = jnp.dot(x1, w1f_ref[...], preferred_element_type=jnp.float32)
    h = h + jnp.dot(x2, w1s_ref[...], preferred_element_type=jnp.float32)
    h = jnp.maximum(h + b1_ref[...], 0.0).astype(jnp.bfloat16)
    y = jnp.dot(h, w2_ref[...], preferred_element_type=jnp.float32) + b2_ref[...]
    mu = jnp.mean(y, axis=-1, keepdims=True)
    var = jnp.mean(jnp.square(y - mu), axis=-1, keepdims=True)
    yn = (y - mu) * lax.rsqrt(var + 1e-5) * g_ref[...] + bb_ref[...]
    rid = lax.broadcasted_iota(jnp.int32, (MOLS_PER_BLK, BLK_ROWS), 0)
    cid = lax.broadcasted_iota(jnp.int32, (MOLS_PER_BLK, BLK_ROWS), 1)
    sel = jnp.where(cid // MOL_SIZE == rid, 1.0 / MOL_SIZE, 0.0)
    pooled_scr[pl.ds(i * MOLS_PER_BLK, MOLS_PER_BLK), :] = jnp.dot(
        sel, yn, preferred_element_type=jnp.float32)

    @pl.when(i == N_BLKS - 1)
    def _():
        p = pooled_scr[...]
        hm = jnp.dot(p, wp_ref[...], preferred_element_type=jnp.float32)
        hm = hm + jnp.dot(f_ref[...], wf_ref[...],
                          preferred_element_type=jnp.float32)
        hm = jnp.maximum(hm + bm1_ref[...], 0.0)
        o = jnp.dot(hm, wm2_ref[...],
                    preferred_element_type=jnp.float32) + bm2_ref[...]
        out_ref[...] = jax.nn.sigmoid(o) * 0.5


def _ffn_pool_tc(f_atoms, aggr, w1f, w1s, b1, w2, b2, g, bb, features, wp, wf,
                 bm1, wm2, bm2):
    return pl.pallas_call(
        _ffn_pool_body,
        grid=(N_BLKS,),
        in_specs=[
            pl.BlockSpec((BLK_ROWS, ATOM_FDIM), lambda i: (i, 0)),
            pl.BlockSpec((BLK_ROWS, HIDDEN), lambda i: (i, 0)),
            pl.BlockSpec((ATOM_FDIM, D_FF), lambda i: (0, 0)),
            pl.BlockSpec((HIDDEN, D_FF), lambda i: (0, 0)),
            pl.BlockSpec((1, D_FF), lambda i: (0, 0)),
            pl.BlockSpec((D_FF, HIDDEN), lambda i: (0, 0)),
            pl.BlockSpec((1, HIDDEN), lambda i: (0, 0)),
            pl.BlockSpec((1, HIDDEN), lambda i: (0, 0)),
            pl.BlockSpec((1, HIDDEN), lambda i: (0, 0)),
            pl.BlockSpec((N_MOLS, FEAT_DIM), lambda i: (0, 0)),
            pl.BlockSpec((HIDDEN, FFN_HID), lambda i: (0, 0)),
            pl.BlockSpec((FEAT_DIM, FFN_HID), lambda i: (0, 0)),
            pl.BlockSpec((1, FFN_HID), lambda i: (0, 0)),
            pl.BlockSpec((FFN_HID, N_TASKS), lambda i: (0, 0)),
            pl.BlockSpec((1, N_TASKS), lambda i: (0, 0)),
        ],
        out_specs=pl.BlockSpec((N_MOLS, N_TASKS), lambda i: (0, 0)),
        out_shape=jax.ShapeDtypeStruct((N_MOLS, N_TASKS), jnp.float32),
        scratch_shapes=[pltpu.VMEM((N_MOLS, HIDDEN), jnp.float32)],
    )(f_atoms, aggr, w1f, w1s, b1, w2, b2, g, bb, features, wp, wf, bm1, wm2,
      bm2)


def kernel(atom_output, bond_output, original_f_atoms, original_f_bonds, a2a,
           a2b, b2a, b2revb, a_scope, features_batch, W1a, b1a, W2a, b2a_w,
           ln1_g, ln1_b, W1b, b1b, W2b, b2b_w, ln2_g, ln2_b, Wm1, bm1, Wm2,
           bm2):
    idx_flat = a2a.reshape(-1)
    idx_flat = jnp.pad(idx_flat, (0, (PAD_ATOMS - N_ATOMS) * MAX_NB))
    idx_rows = idx_flat.reshape(NW, CPW, NIDX)

    aggr = _gather_sum_sc(atom_output, idx_rows)

    out = _ffn_pool_tc(
        original_f_atoms, aggr,
        W1a[:ATOM_FDIM].astype(jnp.bfloat16),
        W1a[ATOM_FDIM:].astype(jnp.bfloat16),
        b1a.reshape(1, D_FF),
        W2a.astype(jnp.bfloat16), b2a_w.reshape(1, HIDDEN),
        ln1_g.reshape(1, HIDDEN), ln1_b.reshape(1, HIDDEN),
        features_batch,
        Wm1[:HIDDEN], Wm1[HIDDEN:], bm1.reshape(1, FFN_HID),
        Wm2, bm2.reshape(1, N_TASKS))
    return out
